# Initial kernel scaffold; baseline (speedup 1.0000x reference)
import jax
import jax.numpy as jnp
from jax.experimental import pallas as pl


def kernel(x, physical_signature, task_context, resource_state, W1, b1, W2, b2, We, be):
    raise NotImplementedError("write your pallas kernel here")


# speedup vs baseline: 2.5463x; 2.5463x over previous
import functools
import math

import jax
import jax.numpy as jnp
from jax.experimental import pallas as pl

B = 4096
IN_DIM = 768
SIG_DIM = 32
E = 8
EXPERT_DIM = 768
HID = E * 2

TB = 512
NB = B // TB

_SQRT2 = math.sqrt(2.0)


def _moe_kernel(x_ref, s2_ref, w1x_ref, w1s_ref, b1_ref, w2_ref, b2_ref,
                we_ref, be_ref, out_ref):
    xb = x_ref[...]
    h = (jnp.dot(xb, w1x_ref[...], preferred_element_type=jnp.float32)
         + jnp.dot(s2_ref[...], w1s_ref[...], preferred_element_type=jnp.float32)
         + b1_ref[...])
    h = 0.5 * h * (1.0 + jax.lax.erf(h / _SQRT2))
    logits = jnp.dot(h, w2_ref[...], preferred_element_type=jnp.float32) + b2_ref[...]
    m = jnp.max(logits, axis=-1, keepdims=True)
    ssum = jnp.sum(jnp.exp(logits - m), axis=-1, keepdims=True)
    w = 1.0 / ssum
    idx = jnp.argmax(logits, axis=-1)[:, None]

    acc = jnp.zeros((TB, EXPERT_DIM), dtype=jnp.float32)
    for j in range(E):
        wj = jnp.where(idx == j, w, 0.0)
        ex = jnp.dot(xb, we_ref[j], preferred_element_type=jnp.float32) + be_ref[j]
        acc = acc + wj * ex
    out_ref[...] = acc


@jax.jit
def kernel(x, physical_signature, task_context, resource_state,
           W1, b1, W2, b2, We, be):
    s2 = jnp.concatenate([physical_signature, task_context, resource_state],
                         axis=-1)
    w1x = W1[:IN_DIM]
    w1s = W1[IN_DIM:]

    grid = (NB,)
    out = pl.pallas_call(
        _moe_kernel,
        grid=grid,
        in_specs=[
            pl.BlockSpec((TB, IN_DIM), lambda i: (i, 0)),
            pl.BlockSpec((TB, SIG_DIM + 3), lambda i: (i, 0)),
            pl.BlockSpec(w1x.shape, lambda i: (0, 0)),
            pl.BlockSpec(w1s.shape, lambda i: (0, 0)),
            pl.BlockSpec((1, HID), lambda i: (0, 0)),
            pl.BlockSpec(W2.shape, lambda i: (0, 0)),
            pl.BlockSpec((1, E), lambda i: (0, 0)),
            pl.BlockSpec(We.shape, lambda i: (0, 0, 0)),
            pl.BlockSpec(be.shape, lambda i: (0, 0)),
        ],
        out_specs=pl.BlockSpec((TB, EXPERT_DIM), lambda i: (i, 0)),
        out_shape=jax.ShapeDtypeStruct((B, EXPERT_DIM), jnp.float32),
    )(x, s2, w1x, w1s, b1[None, :], W2, b2[None, :], We, be)
    return out
